# baseline (speedup 1.0000x reference)
import functools

import jax
import jax.numpy as jnp
from jax import lax
from jax.experimental import pallas as pl
from jax.experimental.pallas import tpu as pltpu
from jax.experimental.pallas import tpu_sc as plsc

LANES = 16


def _proj_body(t_ref, w_ref, o_ref):
    o_ref[...] = jnp.sum(t_ref[...] * w_ref[...], axis=1, keepdims=True)


def _project(table, Wt):
    V, E = table.shape
    BLK = 8192
    return pl.pallas_call(
        _proj_body,
        grid=(V // BLK,),
        in_specs=[
            pl.BlockSpec((BLK, E), lambda i: (i, 0)),
            pl.BlockSpec((1, E), lambda i: (0, 0)),
        ],
        out_specs=pl.BlockSpec((BLK, 1), lambda i: (i, 0)),
        out_shape=jax.ShapeDtypeStruct((V, 1), jnp.float32),
    )(table, Wt)


@functools.lru_cache(maxsize=None)
def _make_sc_pool(B, S):
    info = plsc.get_sparse_core_info()
    NC, NS = info.num_cores, info.num_subcores
    NW = NC * NS
    ROWS_W = B // NW
    CHUNK_ROWS = 128
    CHUNK_IDX = CHUNK_ROWS * S
    NCHUNK = ROWS_W // CHUNK_ROWS

    mesh = plsc.VectorSubcoreMesh(core_axis_name="c", subcore_axis_name="s")

    @functools.partial(
        pl.kernel,
        mesh=mesh,
        out_type=jax.ShapeDtypeStruct((B,), jnp.float32),
        scratch_types=[
            pltpu.VMEM((CHUNK_IDX,), jnp.int32),
            pltpu.VMEM((CHUNK_IDX,), jnp.int32),
            pltpu.VMEM((CHUNK_IDX,), jnp.float32),
            pltpu.VMEM((ROWS_W,), jnp.float32),
            pltpu.VMEM((LANES,), jnp.float32),
            pltpu.VMEM_SHARED((NS * ROWS_W,), jnp.float32),
            pltpu.SemaphoreType.DMA,
        ],
    )
    def sc_pool(tw_hbm, ids_hbm, slots_hbm, b16_hbm, out_hbm,
                idx_v, ridx_v, vals_v, out_v, b_v, acc_sh, sem):
        cid = lax.axis_index("c")
        sid = lax.axis_index("s")
        wid = sid * NC + cid
        row0 = wid * ROWS_W
        pltpu.sync_copy(b16_hbm, b_v)
        bval = b_v[...]
        inv_s = jnp.float32(1.0 / S)
        for g in range(ROWS_W // LANES):
            out_v[pl.ds(g * LANES, LANES)] = jnp.zeros((LANES,), jnp.float32)
        pltpu.sync_copy(out_v, acc_sh.at[pl.ds(sid * ROWS_W, ROWS_W)])
        for c in range(NCHUNK):
            off = (row0 + c * CHUNK_ROWS) * S
            pltpu.sync_copy(ids_hbm.at[pl.ds(off, CHUNK_IDX)], idx_v)
            pltpu.sync_copy(slots_hbm.at[pl.ds(off, CHUNK_IDX)], ridx_v)
            pltpu.async_copy(tw_hbm.at[idx_v], vals_v, sem).wait()
            pltpu.sync_copy(vals_v, acc_sh.at[ridx_v], add=True)
        pltpu.sync_copy(acc_sh.at[pl.ds(sid * ROWS_W, ROWS_W)], out_v)
        for g in range(ROWS_W // LANES):
            r = out_v[pl.ds(g * LANES, LANES)] * inv_s + bval
            out_v[pl.ds(g * LANES, LANES)] = 1.0 / (1.0 + jnp.exp(-r))
        pltpu.sync_copy(out_v, out_hbm.at[pl.ds(row0, ROWS_W)])

    return sc_pool


def kernel(input_ids, table, W, b):
    B, S = input_ids.shape
    V, E = table.shape
    info = plsc.get_sparse_core_info()
    NC, NS = info.num_cores, info.num_subcores
    ROWS_W = B // (NC * NS)
    tw = _project(table, W.reshape(1, E)).reshape(V)
    ids_flat = input_ids.astype(jnp.int32).reshape(B * S)
    row = lax.broadcasted_iota(jnp.int32, (B, S), 0)
    slots = (row // (ROWS_W * NC)) * ROWS_W + row % ROWS_W
    slots_flat = slots.reshape(B * S)
    b16 = jnp.tile(b.astype(jnp.float32).reshape(1), LANES)
    out = _make_sc_pool(B, S)(tw, ids_flat, slots_flat, b16)
    return out.reshape(B, 1)


# speedup vs baseline: 1.1304x; 1.1304x over previous
import functools

import jax
import jax.numpy as jnp
from jax import lax
from jax.experimental import pallas as pl
from jax.experimental.pallas import tpu as pltpu
from jax.experimental.pallas import tpu_sc as plsc

LANES = 16
NSUB = 16
COLS_A = 128


def _proj_body(t_ref, w_ref, o_ref):
    o_ref[...] = jnp.sum(t_ref[...] * w_ref[...], axis=1, keepdims=True)


def _project(table, Wt):
    V, E = table.shape
    BLK = 8192
    return pl.pallas_call(
        _proj_body,
        grid=(V // BLK,),
        in_specs=[
            pl.BlockSpec((BLK, E), lambda i: (i, 0)),
            pl.BlockSpec((1, E), lambda i: (0, 0)),
        ],
        out_specs=pl.BlockSpec((BLK, 1), lambda i: (i, 0)),
        out_shape=jax.ShapeDtypeStruct((V, 1), jnp.float32),
    )(table, Wt)


@functools.lru_cache(maxsize=None)
def _make_sc_pool(B, S):
    info = plsc.get_sparse_core_info()
    NC, NS = info.num_cores, info.num_subcores
    NW = NC * NS
    ROWS_W = B // NW
    CHUNK_ROWS = 128
    NCHUNK = ROWS_W // CHUNK_ROWS
    COLS_B = S - COLS_A
    NA = CHUNK_ROWS * COLS_A
    NB = CHUNK_ROWS * COLS_B

    mesh = plsc.VectorSubcoreMesh(core_axis_name="c", subcore_axis_name="s")

    @functools.partial(
        pl.kernel,
        mesh=mesh,
        out_type=jax.ShapeDtypeStruct((B,), jnp.float32),
        scratch_types=[
            pltpu.VMEM((CHUNK_ROWS, COLS_A), jnp.int32),
            pltpu.VMEM((CHUNK_ROWS, COLS_B), jnp.int32),
            pltpu.VMEM((NA,), jnp.float32),
            pltpu.VMEM((NB,), jnp.float32),
            pltpu.VMEM((NA,), jnp.int32),
            pltpu.VMEM((NB,), jnp.int32),
            pltpu.VMEM((NSUB * CHUNK_ROWS,), jnp.float32),
            pltpu.VMEM((NSUB * CHUNK_ROWS,), jnp.float32),
            pltpu.VMEM((ROWS_W,), jnp.float32),
            pltpu.VMEM((LANES,), jnp.float32),
            pltpu.VMEM_SHARED((NS * NSUB * CHUNK_ROWS,), jnp.float32),
            pltpu.SemaphoreType.DMA,
            pltpu.SemaphoreType.DMA,
        ],
    )
    def sc_pool(tw_hbm, ids_hbm, sa_hbm, sb_hbm, b16_hbm, out_hbm,
                ida_v, idb_v, va_v, vb_v, sa_v, sb_v, acc_v, zero_v,
                out_v, b_v, acc_sh, sem, gsem):
        cid = lax.axis_index("c")
        sid = lax.axis_index("s")
        wid = sid * NC + cid
        row0 = wid * ROWS_W
        pltpu.sync_copy(b16_hbm, b_v)
        bval = b_v[...]
        inv_s = jnp.float32(1.0 / S)
        pltpu.sync_copy(sa_hbm, sa_v)
        pltpu.sync_copy(sb_hbm, sb_v)
        soff = lax.broadcast_in_dim(sid * (NSUB * CHUNK_ROWS), (LANES,), ())
        for j in range(NA // LANES):
            sa_v[pl.ds(j * LANES, LANES)] = (
                sa_v[pl.ds(j * LANES, LANES)] + soff)
        for j in range(NB // LANES):
            sb_v[pl.ds(j * LANES, LANES)] = (
                sb_v[pl.ds(j * LANES, LANES)] + soff)
        for j in range(NSUB * CHUNK_ROWS // LANES):
            zero_v[pl.ds(j * LANES, LANES)] = jnp.zeros((LANES,), jnp.float32)
        my_acc = pl.ds(sid * (NSUB * CHUNK_ROWS), NSUB * CHUNK_ROWS)
        pltpu.sync_copy(zero_v, acc_sh.at[my_acc])
        for c in range(NCHUNK):
            r0 = row0 + c * CHUNK_ROWS
            pltpu.sync_copy(
                ids_hbm.at[pl.ds(r0, CHUNK_ROWS), pl.ds(0, COLS_A)], ida_v)
            pltpu.sync_copy(
                ids_hbm.at[pl.ds(r0, CHUNK_ROWS), pl.ds(COLS_A, COLS_B)],
                idb_v)
            handles = []
            for lr in range(CHUNK_ROWS):
                handles.append(pltpu.async_copy(
                    tw_hbm.at[ida_v.at[lr]],
                    va_v.at[pl.ds(lr * COLS_A, COLS_A)], gsem))
                handles.append(pltpu.async_copy(
                    tw_hbm.at[idb_v.at[lr]],
                    vb_v.at[pl.ds(lr * COLS_B, COLS_B)], gsem))
            for h in handles:
                h.wait()
            pltpu.sync_copy(va_v, acc_sh.at[sa_v], add=True)
            pltpu.sync_copy(vb_v, acc_sh.at[sb_v], add=True)
            pltpu.sync_copy(acc_sh.at[my_acc], acc_v)
            pltpu.sync_copy(zero_v, acc_sh.at[my_acc])
            for g in range(CHUNK_ROWS // LANES):
                acc = acc_v[pl.ds(g * LANES, LANES)]
                for ns in range(1, NSUB):
                    acc = acc + acc_v[pl.ds(ns * CHUNK_ROWS + g * LANES, LANES)]
                r = acc * inv_s + bval
                out_v[pl.ds(c * CHUNK_ROWS + g * LANES, LANES)] = (
                    1.0 / (1.0 + jnp.exp(-r)))
        pltpu.sync_copy(out_v, out_hbm.at[pl.ds(row0, ROWS_W)])

    return sc_pool


def kernel(input_ids, table, W, b):
    B, S = input_ids.shape
    V, E = table.shape
    CHUNK_ROWS = 128
    COLS_B = S - COLS_A
    tw = _project(table, W.reshape(1, E)).reshape(V)
    ja = lax.iota(jnp.int32, CHUNK_ROWS * COLS_A)
    sa = (ja % NSUB) * CHUNK_ROWS + ja // COLS_A
    jb = lax.iota(jnp.int32, CHUNK_ROWS * COLS_B)
    sb = (jb % NSUB) * CHUNK_ROWS + jb // COLS_B
    b16 = jnp.tile(b.astype(jnp.float32).reshape(1), LANES)
    out = _make_sc_pool(B, S)(tw, input_ids.astype(jnp.int32), sa, sb, b16)
    return out.reshape(B, 1)


# speedup vs baseline: 1.5620x; 1.3818x over previous
import functools

import jax
import jax.numpy as jnp
from jax import lax
from jax.experimental import pallas as pl
from jax.experimental.pallas import tpu as pltpu
from jax.experimental.pallas import tpu_sc as plsc

LANES = 16


def _proj_body(t_ref, w_ref, o_ref):
    o_ref[...] = jnp.sum(t_ref[...] * w_ref[...], axis=0)


ROWS3, COLS3, RB = 200, 5000, 8


def _project(tableT, W):
    E, V = tableT.shape
    t3 = tableT.reshape(E, ROWS3, COLS3)
    w3 = W.reshape(E, 1, 1)
    out = pl.pallas_call(
        _proj_body,
        grid=(ROWS3 // RB,),
        in_specs=[
            pl.BlockSpec((E, RB, COLS3), lambda i: (0, i, 0)),
            pl.BlockSpec((E, 1, 1), lambda i: (0, 0, 0)),
        ],
        out_specs=pl.BlockSpec((RB, COLS3), lambda i: (i, 0)),
        out_shape=jax.ShapeDtypeStruct((ROWS3, COLS3), jnp.float32),
    )(t3, w3)
    return out.reshape(V)


@functools.lru_cache(maxsize=None)
def _make_sc_pool(B, S):
    info = plsc.get_sparse_core_info()
    NC, NS = info.num_cores, info.num_subcores
    NW = NC * NS
    ROWS_W = B // NW
    CHUNK_ROWS = 128
    CHUNK_IDX = CHUNK_ROWS * S
    NCHUNK = ROWS_W // CHUNK_ROWS

    mesh = plsc.VectorSubcoreMesh(core_axis_name="c", subcore_axis_name="s")

    @functools.partial(
        pl.kernel,
        mesh=mesh,
        out_type=jax.ShapeDtypeStruct((B,), jnp.float32),
        scratch_types=[
            pltpu.VMEM((CHUNK_IDX,), jnp.int32),
            pltpu.VMEM((CHUNK_IDX,), jnp.float32),
            pltpu.VMEM((ROWS_W,), jnp.float32),
            pltpu.VMEM((LANES,), jnp.float32),
            pltpu.SemaphoreType.DMA,
            pltpu.SemaphoreType.DMA,
        ],
    )
    def sc_pool(tw_hbm, ids_hbm, b16_hbm, out_hbm,
                idx_v, vals_v, out_v, b_v, sem, gsem):
        cid = lax.axis_index("c")
        sid = lax.axis_index("s")
        wid = sid * NC + cid
        row0 = wid * ROWS_W
        pltpu.sync_copy(b16_hbm, b_v)
        bval = b_v[...]
        inv_s = jnp.float32(1.0 / S)
        for c in range(NCHUNK):
            base = row0 + c * CHUNK_ROWS
            handles = []
            for t in range(S):
                handles.append(pltpu.async_copy(
                    ids_hbm.at[pl.ds(t * B + base, CHUNK_ROWS)],
                    idx_v.at[pl.ds(t * CHUNK_ROWS, CHUNK_ROWS)], sem))
            for h in handles:
                h.wait()
            pltpu.async_copy(tw_hbm.at[idx_v], vals_v, gsem).wait()
            for g in range(CHUNK_ROWS // LANES):
                def t_body(t, acc, g=g):
                    return acc + vals_v[pl.ds(t * CHUNK_ROWS + g * LANES,
                                              LANES)]
                acc = lax.fori_loop(
                    0, S, t_body, jnp.zeros((LANES,), jnp.float32))
                r = acc * inv_s + bval
                out_v[pl.ds(c * CHUNK_ROWS + g * LANES, LANES)] = (
                    1.0 / (1.0 + jnp.exp(-r)))
        pltpu.sync_copy(out_v, out_hbm.at[pl.ds(row0, ROWS_W)])

    return sc_pool


def kernel(input_ids, table, W, b):
    B, S = input_ids.shape
    V, E = table.shape
    tw = _project(table.T, W)
    ids_tm = input_ids.astype(jnp.int32).T.reshape(S * B)
    b16 = jnp.tile(b.astype(jnp.float32).reshape(1), LANES)
    out = _make_sc_pool(B, S)(tw, ids_tm, b16)
    return out.reshape(B, 1)


# speedup vs baseline: 3.8198x; 2.4454x over previous
import functools

import jax
import jax.numpy as jnp
from jax import lax
from jax.experimental import pallas as pl
from jax.experimental.pallas import tpu as pltpu
from jax.experimental.pallas import tpu_sc as plsc

LANES = 16


def _proj_body(t_ref, w_ref, o_ref):
    o_ref[...] = jnp.sum(t_ref[...] * w_ref[...], axis=0)


def _project(tableT, W):
    E, V = tableT.shape
    BLKV = 8192
    grid = (V + BLKV - 1) // BLKV
    return pl.pallas_call(
        _proj_body,
        grid=(grid,),
        in_specs=[
            pl.BlockSpec((E, BLKV), lambda i: (0, i)),
            pl.BlockSpec((E, 1), lambda i: (0, 0)),
        ],
        out_specs=pl.BlockSpec((BLKV,), lambda i: (i,)),
        out_shape=jax.ShapeDtypeStruct((V,), jnp.float32),
    )(tableT, W)


@functools.lru_cache(maxsize=None)
def _make_sc_pool(B, S):
    info = plsc.get_sparse_core_info()
    NC, NS = info.num_cores, info.num_subcores
    NW = NC * NS
    ROWS_W = B // NW
    CHUNK_ROWS = 128
    CHUNK_IDX = CHUNK_ROWS * S
    NCHUNK = ROWS_W // CHUNK_ROWS

    mesh = plsc.VectorSubcoreMesh(core_axis_name="c", subcore_axis_name="s")

    @functools.partial(
        pl.kernel,
        mesh=mesh,
        out_type=jax.ShapeDtypeStruct((B,), jnp.float32),
        scratch_types=[
            pltpu.VMEM((CHUNK_IDX,), jnp.int32),
            pltpu.VMEM((CHUNK_IDX,), jnp.float32),
            pltpu.VMEM((ROWS_W,), jnp.float32),
            pltpu.VMEM((LANES,), jnp.float32),
            pltpu.SemaphoreType.DMA,
            pltpu.SemaphoreType.DMA,
        ],
    )
    def sc_pool(tw_hbm, ids_hbm, b16_hbm, out_hbm,
                idx_v, vals_v, out_v, b_v, sem, gsem):
        cid = lax.axis_index("c")
        sid = lax.axis_index("s")
        wid = sid * NC + cid
        row0 = wid * ROWS_W
        pltpu.sync_copy(b16_hbm, b_v)
        bval = b_v[...]
        inv_s = jnp.float32(1.0 / S)
        for c in range(NCHUNK):
            base = row0 + c * CHUNK_ROWS
            handles = []
            for t in range(S):
                handles.append(pltpu.async_copy(
                    ids_hbm.at[pl.ds(t * B + base, CHUNK_ROWS)],
                    idx_v.at[pl.ds(t * CHUNK_ROWS, CHUNK_ROWS)], sem))
            for h in handles:
                h.wait()
            pltpu.async_copy(tw_hbm.at[idx_v], vals_v, gsem).wait()
            for g in range(CHUNK_ROWS // LANES):
                def t_body(t, acc, g=g):
                    return acc + vals_v[pl.ds(t * CHUNK_ROWS + g * LANES,
                                              LANES)]
                acc = lax.fori_loop(
                    0, S, t_body, jnp.zeros((LANES,), jnp.float32))
                r = acc * inv_s + bval
                out_v[pl.ds(c * CHUNK_ROWS + g * LANES, LANES)] = (
                    1.0 / (1.0 + jnp.exp(-r)))
        pltpu.sync_copy(out_v, out_hbm.at[pl.ds(row0, ROWS_W)])

    return sc_pool


def kernel(input_ids, table, W, b):
    B, S = input_ids.shape
    V, E = table.shape
    tw = _project(table.T, W)
    ids_tm = input_ids.astype(jnp.int32).T.reshape(S * B)
    b16 = jnp.tile(b.astype(jnp.float32).reshape(1), LANES)
    out = _make_sc_pool(B, S)(tw, ids_tm, b16)
    return out.reshape(B, 1)


# speedup vs baseline: 4.0641x; 1.0640x over previous
import functools

import jax
import jax.numpy as jnp
from jax import lax
from jax.experimental import pallas as pl
from jax.experimental.pallas import tpu as pltpu
from jax.experimental.pallas import tpu_sc as plsc

LANES = 16


def _proj_body(t_ref, w_ref, o_ref):
    o_ref[...] = jnp.sum(t_ref[...] * w_ref[...], axis=0)


def _project(tableT, W):
    E, V = tableT.shape
    BLKV = 8192
    grid = (V + BLKV - 1) // BLKV
    return pl.pallas_call(
        _proj_body,
        grid=(grid,),
        in_specs=[
            pl.BlockSpec((E, BLKV), lambda i: (0, i)),
            pl.BlockSpec((E, 1), lambda i: (0, 0)),
        ],
        out_specs=pl.BlockSpec((BLKV,), lambda i: (i,)),
        out_shape=jax.ShapeDtypeStruct((V,), jnp.float32),
    )(tableT, W)


@functools.lru_cache(maxsize=None)
def _make_sc_pool(B, S):
    info = plsc.get_sparse_core_info()
    NC, NS = info.num_cores, info.num_subcores
    NW = NC * NS
    ROWS_W = B // NW
    CHUNK_ROWS = 128
    CHUNK_IDX = CHUNK_ROWS * S
    NCHUNK = ROWS_W // CHUNK_ROWS

    mesh = plsc.VectorSubcoreMesh(core_axis_name="c", subcore_axis_name="s")

    @functools.partial(
        pl.kernel,
        mesh=mesh,
        out_type=jax.ShapeDtypeStruct((B,), jnp.float32),
        scratch_types=[
            pltpu.VMEM((CHUNK_IDX,), jnp.int32),
            pltpu.VMEM((CHUNK_IDX,), jnp.int32),
            pltpu.VMEM((CHUNK_IDX,), jnp.float32),
            pltpu.VMEM((CHUNK_IDX,), jnp.float32),
            pltpu.VMEM((ROWS_W,), jnp.float32),
            pltpu.VMEM((LANES,), jnp.float32),
            pltpu.SemaphoreType.DMA,
            pltpu.SemaphoreType.DMA,
        ],
    )
    def sc_pool(tw_hbm, ids_hbm, b16_hbm, out_hbm,
                idx0_v, idx1_v, vals0_v, vals1_v, out_v, b_v, sem, gsem):
        idx_b = (idx0_v, idx1_v)
        vals_b = (vals0_v, vals1_v)
        cid = lax.axis_index("c")
        sid = lax.axis_index("s")
        wid = sid * NC + cid
        row0 = wid * ROWS_W
        pltpu.sync_copy(b16_hbm, b_v)
        bval = b_v[...]
        inv_s = jnp.float32(1.0 / S)

        def stage(c, buf):
            base = row0 + c * CHUNK_ROWS
            return [pltpu.async_copy(
                ids_hbm.at[pl.ds(t * B + base, CHUNK_ROWS)],
                idx_b[buf].at[pl.ds(t * CHUNK_ROWS, CHUNK_ROWS)], sem)
                for t in range(S)]

        def gather(buf):
            return pltpu.async_copy(
                tw_hbm.at[idx_b[buf]], vals_b[buf], gsem)

        for h in stage(0, 0):
            h.wait()
        g_cur = gather(0)
        for c in range(NCHUNK):
            buf = c % 2
            if c + 1 < NCHUNK:
                nxt = stage(c + 1, 1 - buf)
                for h in nxt:
                    h.wait()
                g_nxt = gather(1 - buf)
            g_cur.wait()
            for g in range(CHUNK_ROWS // LANES):
                vv = vals_b[buf]

                def t_body(t, acc, g=g, vv=vv):
                    base = t * (4 * CHUNK_ROWS) + g * LANES
                    a = acc + vv[pl.ds(base, LANES)]
                    a = a + vv[pl.ds(base + CHUNK_ROWS, LANES)]
                    a = a + vv[pl.ds(base + 2 * CHUNK_ROWS, LANES)]
                    a = a + vv[pl.ds(base + 3 * CHUNK_ROWS, LANES)]
                    return a
                acc = lax.fori_loop(
                    0, S // 4, t_body, jnp.zeros((LANES,), jnp.float32))
                r = acc * inv_s + bval
                out_v[pl.ds(c * CHUNK_ROWS + g * LANES, LANES)] = (
                    1.0 / (1.0 + jnp.exp(-r)))
            if c + 1 < NCHUNK:
                g_cur = g_nxt
        pltpu.sync_copy(out_v, out_hbm.at[pl.ds(row0, ROWS_W)])

    return sc_pool


def kernel(input_ids, table, W, b):
    B, S = input_ids.shape
    V, E = table.shape
    tw = _project(table.T, W)
    ids_tm = input_ids.astype(jnp.int32).T.reshape(S * B)
    b16 = jnp.tile(b.astype(jnp.float32).reshape(1), LANES)
    out = _make_sc_pool(B, S)(tw, ids_tm, b16)
    return out.reshape(B, 1)
